# baseline (speedup 1.0000x reference)
import dataclasses
import functools

import jax
import jax.numpy as jnp
from jax import lax
from jax.experimental import pallas as pl
from jax.experimental.pallas import tpu as pltpu
from jax.experimental.pallas import tpu_sc as plsc

N = 10000
NP = 10240
C = 128
E = 320000
NSC = 2
NSUB = 16
NW = NSC * NSUB
CH = 64
NCHT = 160
EPT = NCHT * CH
EP = NW * EPT
RB = 512
NEG_SLOPE_CONST = 0.2


def _prolog_body(x_ref, w_ref, att_ref, xp_ref, al_ref):
    xb = x_ref[...]
    xp = jnp.dot(xb, w_ref[...], preferred_element_type=jnp.float32)
    xp_ref[...] = xp
    al_ref[...] = lax.dot_general(
        att_ref[...], xp, (((1,), (1,)), ((), ())),
        preferred_element_type=jnp.float32)


def _prolog(x_pad, W, att8):
    return pl.pallas_call(
        _prolog_body,
        grid=(NP // RB,),
        in_specs=[
            pl.BlockSpec((RB, C), lambda i: (i, 0)),
            pl.BlockSpec((C, C), lambda i: (0, 0)),
            pl.BlockSpec((8, C), lambda i: (0, 0)),
        ],
        out_specs=[
            pl.BlockSpec((RB, C), lambda i: (i, 0)),
            pl.BlockSpec((8, RB), lambda i: (0, i)),
        ],
        out_shape=[
            jax.ShapeDtypeStruct((NP, C), jnp.float32),
            jax.ShapeDtypeStruct((8, NP), jnp.float32),
        ],
    )(x_pad, W, att8)


NR = 10112
RPT = NR // NSUB


def _sc_a_body(as_hbm, ad_hbm, src_hbm, dst_hbm,
               den_hbm, ex_hbm,
               as_v, ad_v, src_v, dst_v, ex_v, den_v):
    cid = lax.axis_index("c")
    sid = lax.axis_index("s")
    wid = cid * NSUB + sid

    pltpu.sync_copy(as_hbm, as_v)
    pltpu.sync_copy(ad_hbm, ad_v)
    pltpu.sync_copy(src_hbm.at[pl.ds(wid * EPT, EPT)], src_v)
    pltpu.sync_copy(dst_hbm.at[pl.ds(wid * EPT, EPT)], dst_v)

    zero16 = jnp.zeros((16,), jnp.float32)

    @pl.loop(0, NP, step=16)
    def _(i):
        den_v[pl.ds(i, 16)] = zero16

    @pl.loop(0, EPT, step=64)
    def _(i):
        es = []
        for u in range(4):
            s_idx = src_v[pl.ds(i + u * 16, 16)]
            d_idx = dst_v[pl.ds(i + u * 16, 16)]
            a = plsc.load_gather(as_v, [s_idx]) + plsc.load_gather(ad_v, [d_idx])
            a = jnp.maximum(a, a * NEG_SLOPE_CONST)
            es.append((jnp.exp(a), d_idx))
        for u in range(4):
            e, d_idx = es[u]
            ex_v[pl.ds(i + u * 16, 16)] = e
            plsc.addupdate_scatter(den_v, [d_idx], e)

    pltpu.sync_copy(den_v, den_hbm.at[wid])
    pltpu.sync_copy(ex_v, ex_hbm.at[pl.ds(wid * EPT, EPT)])


NB = 5
NCH0 = 160
NCH1 = 160


def _sc_b_body(xp_hbm, src_hbm, dst_hbm, ex_hbm, zeros_hbm,
               outp_hbm,
               src_sl, dst_sl, ex_sl, rows_l, sem_stl, sem_gl, sem_sl,
               out_sh):
    cid = lax.axis_index("c")
    sid = lax.axis_index("s")
    nloc = jnp.where(cid == 0, NCH0, NCH1)
    base = jnp.where(cid == 0, sid * (NCH0 * CH),
                     NSUB * NCH0 * CH + sid * (NCH1 * CH))

    def stage_start(j, b):
        sl = pl.ds(base + j * CH, CH)
        pltpu.async_copy(src_hbm.at[sl], src_sl[b], sem_stl[b])
        pltpu.async_copy(dst_hbm.at[sl], dst_sl[b], sem_stl[b])
        pltpu.async_copy(ex_hbm.at[sl], ex_sl[b], sem_stl[b])

    def stage_wait(j, b):
        sl = pl.ds(base + j * CH, CH)
        pltpu.make_async_copy(src_hbm.at[sl], src_sl[b], sem_stl[b]).wait()
        pltpu.make_async_copy(dst_hbm.at[sl], dst_sl[b], sem_stl[b]).wait()
        pltpu.make_async_copy(ex_hbm.at[sl], ex_sl[b], sem_stl[b]).wait()

    def gather_start(j, b):
        stage_wait(j, b)
        pltpu.async_copy(xp_hbm.at[src_sl[b]], rows_l[b], sem_gl[b])

    def gather_wait(b):
        pltpu.make_async_copy(xp_hbm.at[src_sl[b]], rows_l[b], sem_gl[b]).wait()

    def scatter_start(b):
        pltpu.async_copy(rows_l[b], out_sh.at[dst_sl[b]], sem_sl[b], add=True)

    def scatter_wait(b):
        pltpu.make_async_copy(rows_l[b], out_sh.at[dst_sl[b]], sem_sl[b]).wait()

    with jax.named_scope("b_zero"):
        pltpu.sync_copy(zeros_hbm, out_sh.at[pl.ds(sid * RPT, RPT)])

        stage_start(0, 0)
        stage_start(1, 1)
        stage_start(2, 2)
        gather_start(0, 0)
        gather_start(1, 1)
        plsc.subcore_barrier()

    _loop_scope = jax.named_scope("b_loop")
    _loop_scope.__enter__()

    @pl.loop(0, nloc, step=NB)
    def _(j0):
        for k in range(NB):
            j = j0 + k
            b = k
            b2 = (k + 2) % NB
            b3 = (k + 3) % NB

            @pl.when(j >= 2)
            def _():
                scatter_wait(b3)

            @pl.when(j + 3 < nloc)
            def _():
                stage_start(j + 3, b3)

            @pl.when(j + 2 < nloc)
            def _():
                gather_start(j + 2, b2)

            gather_wait(b)

            @pl.loop(0, CH, step=2)
            def _(r):
                ridx = jnp.full((16,), 0, jnp.int32) + r
                ev0 = plsc.load_gather(ex_sl[b], [ridx])
                ev1 = plsc.load_gather(ex_sl[b], [ridx + 1])
                for c0 in range(0, C, 16):
                    rows_l[b][r, pl.ds(c0, 16)] = rows_l[b][r, pl.ds(c0, 16)] * ev0
                for c0 in range(0, C, 16):
                    rows_l[b][r + 1, pl.ds(c0, 16)] = rows_l[b][r + 1, pl.ds(c0, 16)] * ev1

            scatter_start(b)

    scatter_wait(NB - 2)
    scatter_wait(NB - 1)
    _loop_scope.__exit__(None, None, None)

    with jax.named_scope("b_publish"):
        plsc.subcore_barrier()
        pltpu.sync_copy(out_sh.at[pl.ds(sid * RPT, RPT)],
                        outp_hbm.at[cid, pl.ds(sid * RPT, RPT)])


def _sc_compiler_params():
    cp = pltpu.CompilerParams()
    if "needs_layout_passes" in pltpu.CompilerParams.__dataclass_fields__:
        cp = dataclasses.replace(cp, needs_layout_passes=False)
    if "disable_bounds_checks" in pltpu.CompilerParams.__dataclass_fields__:
        cp = dataclasses.replace(cp, disable_bounds_checks=True)
    return cp


def _sc_stage(as_arr, ad_arr, xp, src_p, dst_p, zeros_b):
    mesh = plsc.VectorSubcoreMesh(core_axis_name="c", subcore_axis_name="s")
    cp = _sc_compiler_params()
    ka = pl.kernel(
        _sc_a_body,
        out_type=[
            jax.ShapeDtypeStruct((NW, NP), jnp.float32),
            jax.ShapeDtypeStruct((EP,), jnp.float32),
        ],
        mesh=mesh,
        compiler_params=cp,
        scratch_types=[
            pltpu.VMEM((NP,), jnp.float32),
            pltpu.VMEM((NP,), jnp.float32),
            pltpu.VMEM((EPT,), jnp.int32),
            pltpu.VMEM((EPT,), jnp.int32),
            pltpu.VMEM((EPT,), jnp.float32),
            pltpu.VMEM((NP,), jnp.float32),
        ],
    )
    denp, ex = ka(as_arr, ad_arr, src_p, dst_p)
    kb = pl.kernel(
        _sc_b_body,
        out_type=jax.ShapeDtypeStruct((NSC, NP, C), jnp.float32),
        mesh=mesh,
        compiler_params=cp,
        scratch_types=[
            [pltpu.VMEM((CH,), jnp.int32)] * NB,
            [pltpu.VMEM((CH,), jnp.int32)] * NB,
            [pltpu.VMEM((CH,), jnp.float32)] * NB,
            [pltpu.VMEM((CH, C), jnp.float32)] * NB,
            [pltpu.SemaphoreType.DMA] * NB,
            [pltpu.SemaphoreType.DMA] * NB,
            [pltpu.SemaphoreType.DMA] * NB,
            pltpu.VMEM_SHARED((NR, C), jnp.float32),
        ],
    )
    parts = kb(xp, src_p, dst_p, ex, zeros_b)
    return parts, denp


def _epilog_body(p_ref, d_ref, b_ref, o_ref):
    s = p_ref[0, :, :] + p_ref[1, :, :]
    ones = jnp.ones((NW, 1), jnp.float32)
    dn = lax.dot_general(d_ref[...], ones, (((0,), (0,)), ((), ())),
                         preferred_element_type=jnp.float32)
    o_ref[...] = s / (dn + 1e-16) + b_ref[...]


def _epilog(parts, denp, bias2):
    return pl.pallas_call(
        _epilog_body,
        grid=(NP // RB,),
        in_specs=[
            pl.BlockSpec((NSC, RB, C), lambda i: (0, i, 0)),
            pl.BlockSpec((NW, RB), lambda i: (0, i)),
            pl.BlockSpec((1, C), lambda i: (0, 0)),
        ],
        out_specs=pl.BlockSpec((RB, C), lambda i: (i, 0)),
        out_shape=jax.ShapeDtypeStruct((NP, C), jnp.float32),
    )(parts, denp, bias2)


def kernel(x, edge_index, W, att_src, att_dst, bias):
    x_pad = jnp.pad(x, ((0, NP - N), (0, 0)))
    att8 = jnp.concatenate(
        [att_src.reshape(1, C), att_dst.reshape(1, C),
         jnp.zeros((6, C), jnp.float32)], axis=0)
    src = edge_index[0]
    dst = edge_index[1]
    pad_idx = N + (jnp.arange(EP - E, dtype=jnp.int32) % 64)
    src_p = jnp.concatenate([src, pad_idx])
    dst_p = jnp.concatenate([dst, pad_idx])
    zeros_b = jnp.zeros((RPT, C), jnp.float32)
    bias2 = bias.reshape(1, C)

    xp, alpha = _prolog(x_pad, W, att8)
    as_arr = alpha[0]
    ad_arr = alpha[1]
    parts, denp = _sc_stage(as_arr, ad_arr, xp, src_p, dst_p, zeros_b)
    out = _epilog(parts, denp, bias2)
    return out[:N]


# speedup vs baseline: 1.0635x; 1.0635x over previous
import dataclasses

import jax
import jax.numpy as jnp
from jax import lax
from jax.experimental import pallas as pl
from jax.experimental.pallas import tpu as pltpu
from jax.experimental.pallas import tpu_sc as plsc

N = 10000
NP = 10240
C = 128
E = 320000
NSC = 2
NSUB = 16
NW = NSC * NSUB
EPT = E // NW
CH = 40
NCH = EPT // CH
RB = 512
RBE = 1024
NEG_SLOPE_CONST = 0.2


def _prolog_body(x_ref, w_ref, att_ref, xp_ref, al_ref):
    xb = x_ref[...]
    xp = jnp.dot(xb, w_ref[...], preferred_element_type=jnp.float32)
    xp_ref[...] = xp
    al_ref[...] = lax.dot_general(
        att_ref[...], xp, (((1,), (1,)), ((), ())),
        preferred_element_type=jnp.float32)


def _prolog(x_pad, W, att8):
    return pl.pallas_call(
        _prolog_body,
        grid=(NP // RB,),
        in_specs=[
            pl.BlockSpec((RB, C), lambda i: (i, 0)),
            pl.BlockSpec((C, C), lambda i: (0, 0)),
            pl.BlockSpec((8, C), lambda i: (0, 0)),
        ],
        out_specs=[
            pl.BlockSpec((RB, C), lambda i: (i, 0)),
            pl.BlockSpec((8, RB), lambda i: (0, i)),
        ],
        out_shape=[
            jax.ShapeDtypeStruct((NP, C), jnp.float32),
            jax.ShapeDtypeStruct((8, NP), jnp.float32),
        ],
    )(x_pad, W, att8)


NR = 10112
RPT = NR // NSUB


def _sc_a_body(al_hbm, ei_hbm,
               den_hbm, ex_hbm,
               as_v, ad_v, src_v, dst_v, ex_v, den_v):
    cid = lax.axis_index("c")
    sid = lax.axis_index("s")
    wid = cid * NSUB + sid

    pltpu.sync_copy(al_hbm.at[0], as_v)
    pltpu.sync_copy(al_hbm.at[1], ad_v)
    pltpu.sync_copy(ei_hbm.at[pl.ds(wid * EPT, EPT)], src_v)
    pltpu.sync_copy(ei_hbm.at[pl.ds(E + wid * EPT, EPT)], dst_v)

    zero16 = jnp.zeros((16,), jnp.float32)

    @pl.loop(0, NP, step=16)
    def _(i):
        den_v[pl.ds(i, 16)] = zero16

    @pl.loop(0, EPT, step=80)
    def _(i):
        es = []
        for u in range(5):
            s_idx = src_v[pl.ds(i + u * 16, 16)]
            d_idx = dst_v[pl.ds(i + u * 16, 16)]
            a = plsc.load_gather(as_v, [s_idx]) + plsc.load_gather(ad_v, [d_idx])
            a = jnp.maximum(a, a * NEG_SLOPE_CONST)
            es.append((jnp.exp(a), d_idx))
        for u in range(5):
            e, d_idx = es[u]
            ex_v[pl.ds(i + u * 16, 16)] = e
            plsc.addupdate_scatter(den_v, [d_idx], e)

    pltpu.sync_copy(den_v, den_hbm.at[wid])
    pltpu.sync_copy(ex_v, ex_hbm.at[pl.ds(wid * EPT, EPT)])


NB = 5


def _sc_b_body(xp_hbm, ei_hbm, ex_hbm,
               outp_hbm,
               src_sl, dst_sl, ex_sl, rows_l, sem_stl, sem_gl, sem_sl,
               out_sh):
    cid = lax.axis_index("c")
    sid = lax.axis_index("s")
    wid = cid * NSUB + sid
    base = wid * EPT

    def stage_start(j, b):
        off = base + j * CH
        pltpu.async_copy(ei_hbm.at[pl.ds(off, CH)], src_sl[b], sem_stl[b])
        pltpu.async_copy(ei_hbm.at[pl.ds(E + off, CH)], dst_sl[b], sem_stl[b])
        pltpu.async_copy(ex_hbm.at[pl.ds(off, CH)], ex_sl[b], sem_stl[b])

    def stage_wait(j, b):
        off = base + j * CH
        pltpu.make_async_copy(ei_hbm.at[pl.ds(off, CH)], src_sl[b], sem_stl[b]).wait()
        pltpu.make_async_copy(ei_hbm.at[pl.ds(E + off, CH)], dst_sl[b], sem_stl[b]).wait()
        pltpu.make_async_copy(ex_hbm.at[pl.ds(off, CH)], ex_sl[b], sem_stl[b]).wait()

    def gather_start(j, b):
        stage_wait(j, b)
        pltpu.async_copy(xp_hbm.at[src_sl[b]], rows_l[b], sem_gl[b])

    def gather_wait(b):
        pltpu.make_async_copy(xp_hbm.at[src_sl[b]], rows_l[b], sem_gl[b]).wait()

    def scatter_start(b):
        pltpu.async_copy(rows_l[b], out_sh.at[dst_sl[b]], sem_sl[b], add=True)

    def scatter_wait(b):
        pltpu.make_async_copy(rows_l[b], out_sh.at[dst_sl[b]], sem_sl[b]).wait()

    with jax.named_scope("b_zero"):
        stage_start(0, 0)
        stage_start(1, 1)
        stage_start(2, 2)

        zero16 = jnp.zeros((16,), jnp.float32)

        @pl.loop(0, CH)
        def _(r):
            for c0 in range(0, C, 16):
                rows_l[0][r, pl.ds(c0, 16)] = zero16

        sbase = sid * RPT
        @pl.loop(0, RPT - CH + 1, step=CH)
        def _(r0):
            pltpu.sync_copy(rows_l[0], out_sh.at[pl.ds(sbase + r0, CH)])
        pltpu.sync_copy(rows_l[0].at[pl.ds(0, RPT % CH)],
                        out_sh.at[pl.ds(sbase + RPT - RPT % CH, RPT % CH)])

        gather_start(0, 0)
        gather_start(1, 1)
        plsc.subcore_barrier()

    _loop_scope = jax.named_scope("b_loop")
    _loop_scope.__enter__()

    @pl.loop(0, NCH, step=NB)
    def _(j0):
        for k in range(NB):
            j = j0 + k
            b = k
            b2 = (k + 2) % NB
            b3 = (k + 3) % NB

            @pl.when(j >= 2)
            def _():
                scatter_wait(b3)

            @pl.when(j + 3 < NCH)
            def _():
                stage_start(j + 3, b3)

            @pl.when(j + 2 < NCH)
            def _():
                gather_start(j + 2, b2)

            gather_wait(b)

            @pl.loop(0, CH, step=2)
            def _(r):
                ridx = jnp.full((16,), 0, jnp.int32) + r
                ev0 = plsc.load_gather(ex_sl[b], [ridx])
                ev1 = plsc.load_gather(ex_sl[b], [ridx + 1])
                for c0 in range(0, C, 16):
                    rows_l[b][r, pl.ds(c0, 16)] = rows_l[b][r, pl.ds(c0, 16)] * ev0
                for c0 in range(0, C, 16):
                    rows_l[b][r + 1, pl.ds(c0, 16)] = rows_l[b][r + 1, pl.ds(c0, 16)] * ev1

            scatter_start(b)

    scatter_wait(NB - 2)
    scatter_wait(NB - 1)
    _loop_scope.__exit__(None, None, None)

    with jax.named_scope("b_publish"):
        plsc.subcore_barrier()
        pltpu.sync_copy(out_sh.at[pl.ds(sid * RPT, RPT)],
                        outp_hbm.at[cid, pl.ds(sid * RPT, RPT)])


def _sc_compiler_params():
    cp = pltpu.CompilerParams()
    if "needs_layout_passes" in pltpu.CompilerParams.__dataclass_fields__:
        cp = dataclasses.replace(cp, needs_layout_passes=False)
    if "disable_bounds_checks" in pltpu.CompilerParams.__dataclass_fields__:
        cp = dataclasses.replace(cp, disable_bounds_checks=True)
    return cp


def _sc_stage(alpha, xp, edge_index):
    mesh = plsc.VectorSubcoreMesh(core_axis_name="c", subcore_axis_name="s")
    cp = _sc_compiler_params()
    ka = pl.kernel(
        _sc_a_body,
        out_type=[
            jax.ShapeDtypeStruct((NW, NP), jnp.float32),
            jax.ShapeDtypeStruct((E,), jnp.float32),
        ],
        mesh=mesh,
        compiler_params=cp,
        scratch_types=[
            pltpu.VMEM((NP,), jnp.float32),
            pltpu.VMEM((NP,), jnp.float32),
            pltpu.VMEM((EPT,), jnp.int32),
            pltpu.VMEM((EPT,), jnp.int32),
            pltpu.VMEM((EPT,), jnp.float32),
            pltpu.VMEM((NP,), jnp.float32),
        ],
    )
    denp, ex = ka(alpha, edge_index)
    kb = pl.kernel(
        _sc_b_body,
        out_type=jax.ShapeDtypeStruct((NSC, NP, C), jnp.float32),
        mesh=mesh,
        compiler_params=cp,
        scratch_types=[
            [pltpu.VMEM((CH,), jnp.int32)] * NB,
            [pltpu.VMEM((CH,), jnp.int32)] * NB,
            [pltpu.VMEM((CH,), jnp.float32)] * NB,
            [pltpu.VMEM((CH, C), jnp.float32)] * NB,
            [pltpu.SemaphoreType.DMA] * NB,
            [pltpu.SemaphoreType.DMA] * NB,
            [pltpu.SemaphoreType.DMA] * NB,
            pltpu.VMEM_SHARED((NR, C), jnp.float32),
        ],
    )
    parts = kb(xp, edge_index, ex)
    return parts, denp


def _epilog_body(p_ref, d_ref, b_ref, o_ref):
    s = p_ref[0, :, :] + p_ref[1, :, :]
    ones = jnp.ones((NW, 1), jnp.float32)
    dn = lax.dot_general(d_ref[...], ones, (((0,), (0,)), ((), ())),
                         preferred_element_type=jnp.float32)
    o_ref[...] = s / (dn + 1e-16) + b_ref[...]


def _epilog(parts, denp, bias2):
    return pl.pallas_call(
        _epilog_body,
        grid=(NP // RBE,),
        in_specs=[
            pl.BlockSpec((NSC, RBE, C), lambda i: (0, i, 0)),
            pl.BlockSpec((NW, RBE), lambda i: (0, i)),
            pl.BlockSpec((1, C), lambda i: (0, 0)),
        ],
        out_specs=pl.BlockSpec((RBE, C), lambda i: (i, 0)),
        out_shape=jax.ShapeDtypeStruct((NP, C), jnp.float32),
    )(parts, denp, bias2)


def kernel(x, edge_index, W, att_src, att_dst, bias):
    x_pad = jnp.pad(x, ((0, NP - N), (0, 0)))
    att8 = jnp.concatenate(
        [att_src.reshape(1, C), att_dst.reshape(1, C),
         jnp.zeros((6, C), jnp.float32)], axis=0)
    bias2 = bias.reshape(1, C)

    xp, alpha = _prolog(x_pad, W, att8)
    parts, denp = _sc_stage(alpha, xp, edge_index.reshape(2 * E))
    return _epilog(parts, denp, bias2)[:N]
